# baseline (speedup 1.0000x reference)
import jax
import jax.numpy as jnp
from jax import lax
from jax.experimental import pallas as pl
from jax.experimental.pallas import tpu as pltpu

FEAT = 512
LATENT = 256
CF = 4
NCB = 4
NEMB = 1024
CDIM = 256
ALPHA = -5.0
B = 32
T = 800
ROWS = (B * T) // NCB
TILE = 640
NT = ROWS // TILE
BIG = 2 ** 30


def _stage_a(x_ref, w1t_ref, z_ref):
    rows = x_ref[...].reshape(T, FEAT)
    z = lax.dot_general(rows.astype(jnp.bfloat16), w1t_ref[...],
                        (((1,), (0,)), ((), ())),
                        preferred_element_type=jnp.float32)
    z_ref[0] = z.reshape(T // CF, NCB * LATENT)


def _stage_b(z_ref, et_ref, e_ref, e2_ref,
             q_ref, kidx_ref, commit_ref, ppls_ref,
             counts_ref, acc_ref):
    i = pl.program_id(0)
    r = pl.program_id(1)

    z = z_ref[...]
    xe = lax.dot_general(z.astype(jnp.bfloat16), et_ref[0],
                         (((1,), (0,)), ((), ())),
                         preferred_element_type=jnp.float32)
    x2 = jnp.sum(z * z, axis=1, keepdims=True)
    d = e2_ref[0] + x2 - 2.0 * xe
    dm = ALPHA * d
    m = jnp.max(dm, axis=1, keepdims=True)
    iota = lax.broadcasted_iota(jnp.int32, (TILE, NEMB), 1)
    k2 = jnp.min(jnp.where(dm == m, iota, BIG), axis=1, keepdims=True)

    hard = (iota == k2).astype(jnp.bfloat16)
    q = lax.dot_general(hard, e_ref[0],
                        (((1,), (0,)), ((), ())),
                        preferred_element_type=jnp.float32)
    q_ref[...] = q
    kidx_ref[...] = jnp.transpose(k2).reshape(1, 1, 1, TILE)

    ones = jnp.full((8, TILE), jnp.bfloat16(1))
    cnt8 = lax.dot_general(ones, hard, (((1,), (0,)), ((), ())),
                           preferred_element_type=jnp.float32)
    cnt = cnt8[0:1]

    @pl.when(r == 0)
    def _():
        counts_ref[...] = cnt

    @pl.when(r > 0)
    def _():
        counts_ref[...] = counts_ref[...] + cnt

    part = jnp.sum((z - q) ** 2)

    @pl.when((i == 0) & (r == 0))
    def _():
        acc_ref[0] = part

    @pl.when((i > 0) | (r > 0))
    def _():
        acc_ref[0] = acc_ref[0] + part

    @pl.when((i == NCB - 1) & (r == NT - 1))
    def _():
        commit_ref[...] = jnp.full((1, 1), acc_ref[0] / (ROWS * CDIM),
                                   jnp.float32)

    @pl.when(r == NT - 1)
    def _():
        probs = counts_ref[...] / float(ROWS)
        ent = -jnp.sum(probs * jnp.log2(probs + 1e-10))
        lane4 = lax.broadcasted_iota(jnp.int32, (1, NCB), 1)
        contrib = jnp.where(lane4 == i, ent, 0.0)
        prev = jnp.where(i == 0, jnp.zeros((1, NCB), jnp.float32),
                         ppls_ref[...])
        ppls_ref[...] = prev + contrib


def _stage_c(q_ref, w2t_ref, out_ref):
    qrows = q_ref[0].reshape(T, LATENT).astype(jnp.bfloat16)
    orows = lax.dot_general(qrows, w2t_ref[...],
                            (((1,), (0,)), ((), ())),
                            preferred_element_type=jnp.float32)
    out_ref[...] = orows.reshape(T * FEAT // 128, 128)


@jax.jit
def _run(x, W1, W2, embeddings):
    w1t = W1.T.astype(jnp.bfloat16)
    w2t = W2.T.astype(jnp.bfloat16)
    et = jnp.transpose(embeddings, (0, 2, 1)).astype(jnp.bfloat16)
    e = embeddings.astype(jnp.bfloat16)
    e2 = jnp.sum(embeddings ** 2, axis=2)[:, None, :]

    z = pl.pallas_call(
        _stage_a,
        grid=(B,),
        in_specs=[
            pl.BlockSpec((T * FEAT // 128, 128), lambda b: (b, 0)),
            pl.BlockSpec((FEAT, LATENT), lambda b: (0, 0)),
        ],
        out_specs=pl.BlockSpec((1, T // CF, NCB * CDIM), lambda b: (b, 0, 0)),
        out_shape=jax.ShapeDtypeStruct((B, T // CF, NCB * CDIM), jnp.float32),
        compiler_params=pltpu.CompilerParams(
            dimension_semantics=("arbitrary",)),
    )(x, w1t)

    zf = z.reshape(ROWS, NCB * CDIM)

    qf, kidx, commit, ppls = pl.pallas_call(
        _stage_b,
        grid=(NCB, NT),
        in_specs=[
            pl.BlockSpec((TILE, CDIM), lambda i, r: (r, i)),
            pl.BlockSpec((1, CDIM, NEMB), lambda i, r: (i, 0, 0)),
            pl.BlockSpec((1, NEMB, CDIM), lambda i, r: (i, 0, 0)),
            pl.BlockSpec((1, 1, NEMB), lambda i, r: (i, 0, 0)),
        ],
        out_specs=[
            pl.BlockSpec((TILE, CDIM), lambda i, r: (r, i)),
            pl.BlockSpec((1, 1, 1, TILE), lambda i, r: (i, r, 0, 0)),
            pl.BlockSpec((1, 1), lambda i, r: (0, 0)),
            pl.BlockSpec((1, NCB), lambda i, r: (0, 0)),
        ],
        out_shape=[
            jax.ShapeDtypeStruct((ROWS, NCB * CDIM), jnp.float32),
            jax.ShapeDtypeStruct((NCB, NT, 1, TILE), jnp.int32),
            jax.ShapeDtypeStruct((1, 1), jnp.float32),
            jax.ShapeDtypeStruct((1, NCB), jnp.float32),
        ],
        scratch_shapes=[
            pltpu.VMEM((1, NEMB), jnp.float32),
            pltpu.SMEM((1,), jnp.float32),
        ],
        compiler_params=pltpu.CompilerParams(
            dimension_semantics=("arbitrary", "arbitrary")),
    )(zf, et, e, e2)

    out = pl.pallas_call(
        _stage_c,
        grid=(B,),
        in_specs=[
            pl.BlockSpec((1, T // CF, NCB * CDIM), lambda b: (b, 0, 0)),
            pl.BlockSpec((LATENT, FEAT), lambda b: (0, 0)),
        ],
        out_specs=pl.BlockSpec((T * FEAT // 128, 128), lambda b: (b, 0)),
        out_shape=jax.ShapeDtypeStruct((B * T * FEAT // 128, 128), jnp.float32),
        compiler_params=pltpu.CompilerParams(
            dimension_semantics=("arbitrary",)),
    )(qf.reshape(B, T // CF, NCB * CDIM), w2t)

    return out, kidx, commit, ppls


def kernel(inputs, W1, W2, embeddings):
    b, c, t, _ = inputs.shape
    x128 = jnp.transpose(inputs[..., 0], (0, 2, 1)).reshape(b * t * c // 128,
                                                            128)
    out128, kidx, commit, ppls = _run(x128, W1, W2, embeddings)
    out = jnp.transpose(out128.reshape(b, t, c), (0, 2, 1))[..., None]
    inds = kidx.reshape(NCB, b, t // CF)
    return out, commit[0, 0], ppls[0], inds


# speedup vs baseline: 1.4140x; 1.4140x over previous
import jax
import jax.numpy as jnp
from jax import lax
from jax.experimental import pallas as pl
from jax.experimental.pallas import tpu as pltpu

FEAT = 512
LATENT = 256
CF = 4
NCB = 4
NEMB = 1024
CDIM = 256
ALPHA = -5.0
B = 32
T = 800
ROWS = (B * T) // NCB
TILE = 640
NT = ROWS // TILE
BIG = 2 ** 30


def _stage_a(x_ref, w1t_ref, z_ref):
    z = lax.dot_general(x_ref[0].astype(jnp.bfloat16), w1t_ref[...],
                        (((1,), (0,)), ((), ())),
                        preferred_element_type=jnp.float32)
    z_ref[0] = z.reshape(T // CF, NCB * LATENT)


def _stage_b(z_ref, et_ref, e_ref, e2_ref,
             q_ref, kidx_ref, commit_ref, ppls_ref,
             counts_ref, acc_ref):
    i = pl.program_id(0)
    r = pl.program_id(1)

    z = z_ref[...]
    xe = lax.dot_general(z.astype(jnp.bfloat16), et_ref[0],
                         (((1,), (0,)), ((), ())),
                         preferred_element_type=jnp.float32)
    x2 = jnp.sum(z * z, axis=1, keepdims=True)
    d = e2_ref[0] + x2 - 2.0 * xe
    dm = ALPHA * d
    m = jnp.max(dm, axis=1, keepdims=True)
    iota = lax.broadcasted_iota(jnp.int32, (TILE, NEMB), 1)
    k2 = jnp.min(jnp.where(dm == m, iota, BIG), axis=1, keepdims=True)

    hard = (iota == k2).astype(jnp.bfloat16)
    q = lax.dot_general(hard, e_ref[0],
                        (((1,), (0,)), ((), ())),
                        preferred_element_type=jnp.float32)
    q_ref[...] = q
    kidx_ref[...] = jnp.transpose(k2).reshape(1, 1, 1, TILE)

    ones = jnp.full((8, TILE), jnp.bfloat16(1))
    cnt8 = lax.dot_general(ones, hard, (((1,), (0,)), ((), ())),
                           preferred_element_type=jnp.float32)
    cnt = cnt8[0:1]

    @pl.when(r == 0)
    def _():
        counts_ref[...] = cnt

    @pl.when(r > 0)
    def _():
        counts_ref[...] = counts_ref[...] + cnt

    part = jnp.sum((z - q) ** 2)

    @pl.when((i == 0) & (r == 0))
    def _():
        acc_ref[0] = part

    @pl.when((i > 0) | (r > 0))
    def _():
        acc_ref[0] = acc_ref[0] + part

    @pl.when((i == NCB - 1) & (r == NT - 1))
    def _():
        commit_ref[...] = jnp.full((1, 1), acc_ref[0] / (ROWS * CDIM),
                                   jnp.float32)

    @pl.when(r == NT - 1)
    def _():
        probs = counts_ref[...] / float(ROWS)
        ent = -jnp.sum(probs * jnp.log2(probs + 1e-10))
        lane4 = lax.broadcasted_iota(jnp.int32, (1, NCB), 1)
        contrib = jnp.where(lane4 == i, ent, 0.0)
        prev = jnp.where(i == 0, jnp.zeros((1, NCB), jnp.float32),
                         ppls_ref[...])
        ppls_ref[...] = prev + contrib


def _stage_c(q_ref, w2t_ref, out_ref):
    qrows = q_ref[0].reshape(T, LATENT).astype(jnp.bfloat16)
    out_ref[0] = lax.dot_general(qrows, w2t_ref[...],
                                 (((1,), (0,)), ((), ())),
                                 preferred_element_type=jnp.float32)


@jax.jit
def _run(x, W1, W2, embeddings):
    w1t = W1.T.astype(jnp.bfloat16)
    w2t = W2.T.astype(jnp.bfloat16)
    et = jnp.transpose(embeddings, (0, 2, 1)).astype(jnp.bfloat16)
    e = embeddings.astype(jnp.bfloat16)
    e2 = jnp.sum(embeddings ** 2, axis=2)[:, None, :]

    z = pl.pallas_call(
        _stage_a,
        grid=(B,),
        in_specs=[
            pl.BlockSpec((1, T, FEAT), lambda b: (b, 0, 0)),
            pl.BlockSpec((FEAT, LATENT), lambda b: (0, 0)),
        ],
        out_specs=pl.BlockSpec((1, T // CF, NCB * CDIM), lambda b: (b, 0, 0)),
        out_shape=jax.ShapeDtypeStruct((B, T // CF, NCB * CDIM), jnp.float32),
        compiler_params=pltpu.CompilerParams(
            dimension_semantics=("arbitrary",)),
    )(x, w1t)

    zf = z.reshape(ROWS, NCB * CDIM)

    qf, kidx, commit, ppls = pl.pallas_call(
        _stage_b,
        grid=(NCB, NT),
        in_specs=[
            pl.BlockSpec((TILE, CDIM), lambda i, r: (r, i)),
            pl.BlockSpec((1, CDIM, NEMB), lambda i, r: (i, 0, 0)),
            pl.BlockSpec((1, NEMB, CDIM), lambda i, r: (i, 0, 0)),
            pl.BlockSpec((1, 1, NEMB), lambda i, r: (i, 0, 0)),
        ],
        out_specs=[
            pl.BlockSpec((TILE, CDIM), lambda i, r: (r, i)),
            pl.BlockSpec((1, 1, 1, TILE), lambda i, r: (i, r, 0, 0)),
            pl.BlockSpec((1, 1), lambda i, r: (0, 0)),
            pl.BlockSpec((1, NCB), lambda i, r: (0, 0)),
        ],
        out_shape=[
            jax.ShapeDtypeStruct((ROWS, NCB * CDIM), jnp.float32),
            jax.ShapeDtypeStruct((NCB, NT, 1, TILE), jnp.int32),
            jax.ShapeDtypeStruct((1, 1), jnp.float32),
            jax.ShapeDtypeStruct((1, NCB), jnp.float32),
        ],
        scratch_shapes=[
            pltpu.VMEM((1, NEMB), jnp.float32),
            pltpu.SMEM((1,), jnp.float32),
        ],
        compiler_params=pltpu.CompilerParams(
            dimension_semantics=("arbitrary", "arbitrary")),
    )(zf, et, e, e2)

    out = pl.pallas_call(
        _stage_c,
        grid=(B,),
        in_specs=[
            pl.BlockSpec((1, T // CF, NCB * CDIM), lambda b: (b, 0, 0)),
            pl.BlockSpec((LATENT, FEAT), lambda b: (0, 0)),
        ],
        out_specs=pl.BlockSpec((1, T, FEAT), lambda b: (b, 0, 0)),
        out_shape=jax.ShapeDtypeStruct((B, T, FEAT), jnp.float32),
        compiler_params=pltpu.CompilerParams(
            dimension_semantics=("arbitrary",)),
    )(qf.reshape(B, T // CF, NCB * CDIM), w2t)

    return out, kidx, commit, ppls


def kernel(inputs, W1, W2, embeddings):
    b, c, t, _ = inputs.shape
    xt = jnp.transpose(inputs[..., 0], (0, 2, 1))
    out3, kidx, commit, ppls = _run(xt, W1, W2, embeddings)
    out = jnp.transpose(out3, (0, 2, 1))[..., None]
    inds = kidx.reshape(NCB, b, t // CF)
    return out, commit[0, 0], ppls[0], inds


# speedup vs baseline: 1.5116x; 1.0691x over previous
import jax
import jax.numpy as jnp
from jax import lax
from jax.experimental import pallas as pl
from jax.experimental.pallas import tpu as pltpu

FEAT = 512
LATENT = 256
CF = 4
NCB = 4
NEMB = 1024
CDIM = 256
ALPHA = -5.0
B = 32
T = 800
ROWS = (B * T) // NCB
TILE = 3200
NT = ROWS // TILE
BIG = 2 ** 30


def _stage_a(x_ref, w1t_ref, z_ref):
    z = lax.dot_general(x_ref[0].astype(jnp.bfloat16), w1t_ref[...],
                        (((1,), (0,)), ((), ())),
                        preferred_element_type=jnp.float32)
    z_ref[0] = z.reshape(T // CF, NCB * LATENT)


def _stage_b(z_ref, et_ref, e_ref, e2_ref,
             q_ref, kidx_ref, commit_ref, ppls_ref,
             counts_ref, acc_ref):
    i = pl.program_id(0)
    r = pl.program_id(1)

    z = z_ref[...]
    xe = lax.dot_general(z.astype(jnp.bfloat16), et_ref[0],
                         (((1,), (0,)), ((), ())),
                         preferred_element_type=jnp.float32)
    x2 = jnp.sum(z * z, axis=1, keepdims=True)
    d = e2_ref[0] + x2 - 2.0 * xe
    dm = ALPHA * d
    m = jnp.max(dm, axis=1, keepdims=True)
    iota = lax.broadcasted_iota(jnp.int32, (TILE, NEMB), 1)
    k2 = jnp.min(jnp.where(dm == m, iota, BIG), axis=1, keepdims=True)

    hard = (iota == k2).astype(jnp.bfloat16)
    q = lax.dot_general(hard, e_ref[0],
                        (((1,), (0,)), ((), ())),
                        preferred_element_type=jnp.float32)
    q_ref[...] = q
    kidx_ref[...] = jnp.transpose(k2).reshape(1, 1, 1, TILE)

    ones = jnp.full((8, TILE), jnp.bfloat16(1))
    cnt8 = lax.dot_general(ones, hard, (((1,), (0,)), ((), ())),
                           preferred_element_type=jnp.float32)
    cnt = cnt8[0:1]

    @pl.when(r == 0)
    def _():
        counts_ref[...] = cnt

    @pl.when(r > 0)
    def _():
        counts_ref[...] = counts_ref[...] + cnt

    part = jnp.sum((z - q) ** 2)

    @pl.when((i == 0) & (r == 0))
    def _():
        acc_ref[0] = part

    @pl.when((i > 0) | (r > 0))
    def _():
        acc_ref[0] = acc_ref[0] + part

    @pl.when((i == NCB - 1) & (r == NT - 1))
    def _():
        commit_ref[...] = jnp.full((1, 1), acc_ref[0] / (ROWS * CDIM),
                                   jnp.float32)

    @pl.when(r == NT - 1)
    def _():
        probs = counts_ref[...] / float(ROWS)
        ent = -jnp.sum(probs * jnp.log2(probs + 1e-10))
        lane4 = lax.broadcasted_iota(jnp.int32, (1, NCB), 1)
        contrib = jnp.where(lane4 == i, ent, 0.0)
        prev = jnp.where(i == 0, jnp.zeros((1, NCB), jnp.float32),
                         ppls_ref[...])
        ppls_ref[...] = prev + contrib


def _stage_c(q_ref, w2t_ref, out_ref):
    qrows = q_ref[0].reshape(T, LATENT).astype(jnp.bfloat16)
    out_ref[0] = lax.dot_general(qrows, w2t_ref[...],
                                 (((1,), (0,)), ((), ())),
                                 preferred_element_type=jnp.float32)


@jax.jit
def _run(x, W1, W2, embeddings):
    w1t = W1.T.astype(jnp.bfloat16)
    w2t = W2.T.astype(jnp.bfloat16)
    et = jnp.transpose(embeddings, (0, 2, 1)).astype(jnp.bfloat16)
    e = embeddings.astype(jnp.bfloat16)
    e2 = jnp.sum(embeddings ** 2, axis=2)[:, None, :]

    z = pl.pallas_call(
        _stage_a,
        grid=(B,),
        in_specs=[
            pl.BlockSpec((1, T, FEAT), lambda b: (b, 0, 0)),
            pl.BlockSpec((FEAT, LATENT), lambda b: (0, 0)),
        ],
        out_specs=pl.BlockSpec((1, T // CF, NCB * CDIM), lambda b: (b, 0, 0)),
        out_shape=jax.ShapeDtypeStruct((B, T // CF, NCB * CDIM), jnp.float32),
        compiler_params=pltpu.CompilerParams(
            dimension_semantics=("arbitrary",)),
    )(x, w1t)

    zf = z.reshape(ROWS, NCB * CDIM)

    qf, kidx, commit, ppls = pl.pallas_call(
        _stage_b,
        grid=(NCB, NT),
        in_specs=[
            pl.BlockSpec((TILE, CDIM), lambda i, r: (r, i)),
            pl.BlockSpec((1, CDIM, NEMB), lambda i, r: (i, 0, 0)),
            pl.BlockSpec((1, NEMB, CDIM), lambda i, r: (i, 0, 0)),
            pl.BlockSpec((1, 1, NEMB), lambda i, r: (i, 0, 0)),
        ],
        out_specs=[
            pl.BlockSpec((TILE, CDIM), lambda i, r: (r, i)),
            pl.BlockSpec((1, 1, 1, TILE), lambda i, r: (i, r, 0, 0)),
            pl.BlockSpec((1, 1), lambda i, r: (0, 0)),
            pl.BlockSpec((1, NCB), lambda i, r: (0, 0)),
        ],
        out_shape=[
            jax.ShapeDtypeStruct((ROWS, NCB * CDIM), jnp.float32),
            jax.ShapeDtypeStruct((NCB, NT, 1, TILE), jnp.int32),
            jax.ShapeDtypeStruct((1, 1), jnp.float32),
            jax.ShapeDtypeStruct((1, NCB), jnp.float32),
        ],
        scratch_shapes=[
            pltpu.VMEM((1, NEMB), jnp.float32),
            pltpu.SMEM((1,), jnp.float32),
        ],
        compiler_params=pltpu.CompilerParams(
            dimension_semantics=("arbitrary", "arbitrary")),
    )(zf, et, e, e2)

    out = pl.pallas_call(
        _stage_c,
        grid=(B,),
        in_specs=[
            pl.BlockSpec((1, T // CF, NCB * CDIM), lambda b: (b, 0, 0)),
            pl.BlockSpec((LATENT, FEAT), lambda b: (0, 0)),
        ],
        out_specs=pl.BlockSpec((1, T, FEAT), lambda b: (b, 0, 0)),
        out_shape=jax.ShapeDtypeStruct((B, T, FEAT), jnp.float32),
        compiler_params=pltpu.CompilerParams(
            dimension_semantics=("arbitrary",)),
    )(qf.reshape(B, T // CF, NCB * CDIM), w2t)

    return out, kidx, commit, ppls


def kernel(inputs, W1, W2, embeddings):
    b, c, t, _ = inputs.shape
    xt = jnp.transpose(inputs[..., 0], (0, 2, 1))
    out3, kidx, commit, ppls = _run(xt, W1, W2, embeddings)
    out = jnp.transpose(out3, (0, 2, 1))[..., None]
    inds = kidx.reshape(NCB, b, t // CF)
    return out, commit[0, 0], ppls[0], inds


# speedup vs baseline: 1.5654x; 1.0356x over previous
import jax
import jax.numpy as jnp
from jax import lax
from jax.experimental import pallas as pl
from jax.experimental.pallas import tpu as pltpu

FEAT = 512
LATENT = 256
CF = 4
NCB = 4
NEMB = 1024
CDIM = 256
ALPHA = -5.0
B = 32
T = 800
ROWS = (B * T) // NCB
TILE = 3200
NT = ROWS // TILE
BIG = 2 ** 30
TPB = T // CF


def _fused_ab(x_ref, w1t_ref, et_ref, e_ref, e2_ref,
              qf_ref, kidx_ref, commit_ref, ppls_ref,
              z_scr, counts_ref, acc_ref):
    s = pl.program_id(0)

    @pl.when(s < B)
    def _():
        z = lax.dot_general(x_ref[0].astype(jnp.bfloat16), w1t_ref[...],
                            (((1,), (0,)), ((), ())),
                            preferred_element_type=jnp.float32)
        z_scr[pl.ds(s * TPB, TPB), :] = z.reshape(TPB, NCB * LATENT)

    @pl.when(s >= B)
    def _():
        t = s - B
        i = t // NT
        r = t % NT

        z = z_scr[pl.ds(r * TILE, TILE), pl.ds(i * CDIM, CDIM)]
        xe = lax.dot_general(z.astype(jnp.bfloat16), et_ref[0],
                             (((1,), (0,)), ((), ())),
                             preferred_element_type=jnp.float32)
        x2 = jnp.sum(z * z, axis=1, keepdims=True)
        d = e2_ref[0] + x2 - 2.0 * xe
        dm = ALPHA * d
        m = jnp.max(dm, axis=1, keepdims=True)
        iota = lax.broadcasted_iota(jnp.int32, (TILE, NEMB), 1)
        k2 = jnp.min(jnp.where(dm == m, iota, BIG), axis=1, keepdims=True)

        hard = (iota == k2).astype(jnp.bfloat16)
        q = lax.dot_general(hard, e_ref[0],
                            (((1,), (0,)), ((), ())),
                            preferred_element_type=jnp.float32)
        qf_ref[...] = q
        kidx_ref[...] = jnp.transpose(k2).reshape(1, 1, 1, TILE)

        ones = jnp.full((8, TILE), jnp.bfloat16(1))
        cnt8 = lax.dot_general(ones, hard, (((1,), (0,)), ((), ())),
                               preferred_element_type=jnp.float32)
        cnt = cnt8[0:1]

        @pl.when(r == 0)
        def _():
            counts_ref[...] = cnt

        @pl.when(r > 0)
        def _():
            counts_ref[...] = counts_ref[...] + cnt

        part = jnp.sum((z - q) ** 2)

        @pl.when(t == 0)
        def _():
            acc_ref[0] = part

        @pl.when(t > 0)
        def _():
            acc_ref[0] = acc_ref[0] + part

        @pl.when(t == NCB * NT - 1)
        def _():
            commit_ref[...] = jnp.full((1, 1), acc_ref[0] / (ROWS * CDIM),
                                       jnp.float32)

        @pl.when(r == NT - 1)
        def _():
            probs = counts_ref[...] / float(ROWS)
            ent = -jnp.sum(probs * jnp.log2(probs + 1e-10))
            lane4 = lax.broadcasted_iota(jnp.int32, (1, NCB), 1)
            contrib = jnp.where(lane4 == i, ent, 0.0)
            prev = jnp.where(i == 0, jnp.zeros((1, NCB), jnp.float32),
                             ppls_ref[...])
            ppls_ref[...] = prev + contrib


def _stage_c(q_ref, w2t_ref, out_ref):
    qrows = q_ref[0].reshape(T, LATENT).astype(jnp.bfloat16)
    out_ref[0] = lax.dot_general(qrows, w2t_ref[...],
                                 (((1,), (0,)), ((), ())),
                                 preferred_element_type=jnp.float32)


def _bi(s):
    return jnp.maximum(s - B, 0)


@jax.jit
def _run(x, W1, W2, embeddings):
    w1t = W1.T.astype(jnp.bfloat16)
    w2t = W2.T.astype(jnp.bfloat16)
    et = jnp.transpose(embeddings, (0, 2, 1)).astype(jnp.bfloat16)
    e = embeddings.astype(jnp.bfloat16)
    e2 = jnp.sum(embeddings ** 2, axis=2)[:, None, :]

    qf, kidx, commit, ppls = pl.pallas_call(
        _fused_ab,
        grid=(B + NCB * NT,),
        in_specs=[
            pl.BlockSpec((1, T, FEAT), lambda s: (jnp.minimum(s, B - 1), 0, 0)),
            pl.BlockSpec((FEAT, LATENT), lambda s: (0, 0)),
            pl.BlockSpec((1, CDIM, NEMB), lambda s: (_bi(s) // NT, 0, 0)),
            pl.BlockSpec((1, NEMB, CDIM), lambda s: (_bi(s) // NT, 0, 0)),
            pl.BlockSpec((1, 1, NEMB), lambda s: (_bi(s) // NT, 0, 0)),
        ],
        out_specs=[
            pl.BlockSpec((TILE, CDIM), lambda s: (_bi(s) % NT, _bi(s) // NT)),
            pl.BlockSpec((1, 1, 1, TILE),
                         lambda s: (_bi(s) // NT, _bi(s) % NT, 0, 0)),
            pl.BlockSpec((1, 1), lambda s: (0, 0)),
            pl.BlockSpec((1, NCB), lambda s: (0, 0)),
        ],
        out_shape=[
            jax.ShapeDtypeStruct((ROWS, NCB * CDIM), jnp.float32),
            jax.ShapeDtypeStruct((NCB, NT, 1, TILE), jnp.int32),
            jax.ShapeDtypeStruct((1, 1), jnp.float32),
            jax.ShapeDtypeStruct((1, NCB), jnp.float32),
        ],
        scratch_shapes=[
            pltpu.VMEM((ROWS, NCB * CDIM), jnp.float32),
            pltpu.VMEM((1, NEMB), jnp.float32),
            pltpu.SMEM((1,), jnp.float32),
        ],
        compiler_params=pltpu.CompilerParams(
            dimension_semantics=("arbitrary",)),
    )(x, w1t, et, e, e2)

    out = pl.pallas_call(
        _stage_c,
        grid=(B,),
        in_specs=[
            pl.BlockSpec((1, T // CF, NCB * CDIM), lambda b: (b, 0, 0)),
            pl.BlockSpec((LATENT, FEAT), lambda b: (0, 0)),
        ],
        out_specs=pl.BlockSpec((1, T, FEAT), lambda b: (b, 0, 0)),
        out_shape=jax.ShapeDtypeStruct((B, T, FEAT), jnp.float32),
        compiler_params=pltpu.CompilerParams(
            dimension_semantics=("arbitrary",)),
    )(qf.reshape(B, T // CF, NCB * CDIM), w2t)

    return out, kidx, commit, ppls


def kernel(inputs, W1, W2, embeddings):
    b, c, t, _ = inputs.shape
    xt = jnp.transpose(inputs[..., 0], (0, 2, 1))
    out3, kidx, commit, ppls = _run(xt, W1, W2, embeddings)
    out = jnp.transpose(out3, (0, 2, 1))[..., None]
    inds = kidx.reshape(NCB, b, t // CF)
    return out, commit[0, 0], ppls[0], inds


# speedup vs baseline: 1.5683x; 1.0019x over previous
import jax
import jax.numpy as jnp
from jax import lax
from jax.experimental import pallas as pl
from jax.experimental.pallas import tpu as pltpu

FEAT = 512
LATENT = 256
CF = 4
NCB = 4
NEMB = 1024
CDIM = 256
ALPHA = -5.0
B = 32
T = 800
ROWS = (B * T) // NCB
TILE = 3200
NT = ROWS // TILE
BIG = 2 ** 30
TPB = T // CF


def _fused_ab(x_ref, w1t_ref, et_ref, e_ref, e2_ref,
              qf_ref, kidx_ref, commit_ref, ppls_ref,
              z_scr, counts_ref, acc_ref):
    s = pl.program_id(0)

    @pl.when(s < B)
    def _():
        z = lax.dot_general(x_ref[0].astype(jnp.bfloat16), w1t_ref[...],
                            (((1,), (0,)), ((), ())),
                            preferred_element_type=jnp.float32)
        z_scr[pl.ds(s * TPB, TPB), :] = z.reshape(TPB, NCB * LATENT)

    @pl.when(s >= B)
    def _():
        t = s - B
        i = t // NT
        r = t % NT

        z = z_scr[pl.ds(r * TILE, TILE), pl.ds(i * CDIM, CDIM)]
        xe = lax.dot_general(z.astype(jnp.bfloat16), et_ref[0],
                             (((1,), (0,)), ((), ())),
                             preferred_element_type=jnp.float32)
        x2 = jnp.sum(z * z, axis=1, keepdims=True)
        d = e2_ref[0] + x2 - 2.0 * xe
        dm = ALPHA * d
        m = jnp.max(dm, axis=1, keepdims=True)
        iota = lax.broadcasted_iota(jnp.int32, (TILE, NEMB), 1)
        k2 = jnp.min(jnp.where(dm == m, iota, BIG), axis=1, keepdims=True)

        hard = (iota == k2).astype(jnp.bfloat16)
        q = lax.dot_general(hard, e_ref[0],
                            (((1,), (0,)), ((), ())),
                            preferred_element_type=jnp.float32)
        qf_ref[...] = q
        kidx_ref[...] = jnp.transpose(k2).reshape(1, 1, 1, TILE)

        ones = jnp.full((8, TILE), jnp.bfloat16(1))
        cnt8 = lax.dot_general(ones, hard, (((1,), (0,)), ((), ())),
                               preferred_element_type=jnp.float32)
        cnt = cnt8[0:1]

        @pl.when(r == 0)
        def _():
            counts_ref[...] = cnt

        @pl.when(r > 0)
        def _():
            counts_ref[...] = counts_ref[...] + cnt

        part = jnp.sum((z - q) ** 2)

        @pl.when(t == 0)
        def _():
            acc_ref[0] = part

        @pl.when(t > 0)
        def _():
            acc_ref[0] = acc_ref[0] + part

        @pl.when(t == NCB * NT - 1)
        def _():
            commit_ref[...] = jnp.full((1, 1), acc_ref[0] / (ROWS * CDIM),
                                       jnp.float32)

        @pl.when(r == NT - 1)
        def _():
            probs = counts_ref[...] / float(ROWS)
            ent = -jnp.sum(probs * jnp.log2(probs + 1e-10))
            lane4 = lax.broadcasted_iota(jnp.int32, (1, NCB), 1)
            contrib = jnp.where(lane4 == i, ent, 0.0)
            prev = jnp.where(i == 0, jnp.zeros((1, NCB), jnp.float32),
                             ppls_ref[...])
            ppls_ref[...] = prev + contrib


def _stage_c(q_ref, w2t_ref, out_ref):
    qrows = q_ref[0].reshape(T, LATENT).astype(jnp.bfloat16)
    orows = lax.dot_general(qrows, w2t_ref[...],
                            (((1,), (0,)), ((), ())),
                            preferred_element_type=jnp.float32)
    out_ref[0, 0] = orows


def _bi(s):
    return jnp.maximum(s - B, 0)


@jax.jit
def _run(x, W1, W2, embeddings):
    w1t = W1.T.astype(jnp.bfloat16)
    w2t = W2.T.astype(jnp.bfloat16)
    et = jnp.transpose(embeddings, (0, 2, 1)).astype(jnp.bfloat16)
    e = embeddings.astype(jnp.bfloat16)
    e2 = jnp.sum(embeddings ** 2, axis=2)[:, None, :]

    qf, kidx, commit, ppls = pl.pallas_call(
        _fused_ab,
        grid=(B + NCB * NT,),
        in_specs=[
            pl.BlockSpec((1, T, FEAT), lambda s: (jnp.minimum(s, B - 1), 0, 0)),
            pl.BlockSpec((FEAT, LATENT), lambda s: (0, 0)),
            pl.BlockSpec((1, CDIM, NEMB), lambda s: (_bi(s) // NT, 0, 0)),
            pl.BlockSpec((1, NEMB, CDIM), lambda s: (_bi(s) // NT, 0, 0)),
            pl.BlockSpec((1, 1, NEMB), lambda s: (_bi(s) // NT, 0, 0)),
        ],
        out_specs=[
            pl.BlockSpec((TILE, CDIM), lambda s: (_bi(s) % NT, _bi(s) // NT)),
            pl.BlockSpec((1, 1, 1, TILE),
                         lambda s: (_bi(s) // NT, _bi(s) % NT, 0, 0)),
            pl.BlockSpec((1, 1), lambda s: (0, 0)),
            pl.BlockSpec((1, NCB), lambda s: (0, 0)),
        ],
        out_shape=[
            jax.ShapeDtypeStruct((ROWS, NCB * CDIM), jnp.float32),
            jax.ShapeDtypeStruct((NCB, NT, 1, TILE), jnp.int32),
            jax.ShapeDtypeStruct((1, 1), jnp.float32),
            jax.ShapeDtypeStruct((1, NCB), jnp.float32),
        ],
        scratch_shapes=[
            pltpu.VMEM((ROWS, NCB * CDIM), jnp.float32),
            pltpu.VMEM((1, NEMB), jnp.float32),
            pltpu.SMEM((1,), jnp.float32),
        ],
        compiler_params=pltpu.CompilerParams(
            dimension_semantics=("arbitrary",)),
    )(x, w1t, et, e, e2)

    out = pl.pallas_call(
        _stage_c,
        grid=(B,),
        in_specs=[
            pl.BlockSpec((1, T // CF, NCB * CDIM), lambda b: (b, 0, 0)),
            pl.BlockSpec((LATENT, FEAT), lambda b: (0, 0)),
        ],
        out_specs=pl.BlockSpec((1, 1, T, FEAT), lambda b: (b, 0, 0, 0)),
        out_shape=jax.ShapeDtypeStruct((B, 1, T, FEAT), jnp.float32),
        compiler_params=pltpu.CompilerParams(
            dimension_semantics=("arbitrary",)),
    )(qf.reshape(B, T // CF, NCB * CDIM), w2t)

    return out, kidx, commit, ppls


def kernel(inputs, W1, W2, embeddings):
    b, c, t, _ = inputs.shape
    xt = jnp.transpose(inputs[..., 0], (0, 2, 1))
    out4, kidx, commit, ppls = _run(xt, W1, W2, embeddings)
    out = jnp.transpose(out4, (0, 3, 2, 1))
    inds = kidx.reshape(NCB, b, t // CF)
    return out, commit[0, 0], ppls[0], inds
